# baseline (device time: 21225 ns/iter reference)
import jax
import jax.numpy as jnp
from jax import lax
from jax.experimental import pallas as pl
from jax.experimental.pallas import tpu as pltpu

N_DEV = 4
N_LAYERS = 3
B = 128
D = 128
H = 256


def kernel(x, Win0, Wout0, Win1, Wout1, Win2, Wout2):
    def body(x_ref, win0_ref, wout0_ref, win1_ref, wout1_ref,
             win2_ref, wout2_ref, out_ref,
             partial_ref, comm_ref, acc_ref, send_sems, recv_sems):
        my = lax.axis_index("i")

        barrier_sem = pltpu.get_barrier_semaphore()
        for d in range(1, N_DEV):
            peer = lax.rem(my + d, N_DEV)
            pl.semaphore_signal(
                barrier_sem, inc=1,
                device_id=(peer,), device_id_type=pl.DeviceIdType.MESH,
            )
        pl.semaphore_wait(barrier_sem, N_DEV - 1)

        win_refs = [win0_ref, win1_ref, win2_ref]
        wout_refs = [wout0_ref, wout1_ref, wout2_ref]

        xv = x_ref[:, :]
        for r in range(N_LAYERS):
            h = jnp.maximum(
                jnp.dot(xv, win_refs[r][:, :],
                        preferred_element_type=jnp.float32),
                0.0,
            )
            partial = jnp.dot(h, wout_refs[r][:, :],
                              preferred_element_type=jnp.float32)
            partial_ref[r] = partial

            rdmas = []
            for d in range(1, N_DEV):
                peer = lax.rem(my + d, N_DEV)
                rdma = pltpu.make_async_remote_copy(
                    src_ref=partial_ref.at[r],
                    dst_ref=comm_ref.at[r, d - 1],
                    send_sem=send_sems.at[r, d - 1],
                    recv_sem=recv_sems.at[r, d - 1],
                    device_id=(peer,),
                    device_id_type=pl.DeviceIdType.MESH,
                )
                rdma.start()
                rdmas.append(rdma)
            for rdma in rdmas:
                rdma.wait_recv()
            acc = partial
            for d in range(1, N_DEV):
                acc = acc + comm_ref[r, d - 1]
            for rdma in rdmas:
                rdma.wait_send()
            xv = acc

        acc_ref[:, :] = xv
        out_ref[:, :] = acc_ref[pl.ds(my * (B // N_DEV), B // N_DEV), :]

    return pl.pallas_call(
        body,
        out_shape=jax.ShapeDtypeStruct((B // N_DEV, D), jnp.float32),
        in_specs=[pl.BlockSpec(memory_space=pltpu.VMEM)] * 7,
        out_specs=pl.BlockSpec(memory_space=pltpu.VMEM),
        scratch_shapes=[
            pltpu.VMEM((N_LAYERS, B, D), jnp.float32),
            pltpu.VMEM((N_LAYERS, N_DEV - 1, B, D), jnp.float32),
            pltpu.VMEM((B, D), jnp.float32),
            pltpu.SemaphoreType.DMA((N_LAYERS, N_DEV - 1)),
            pltpu.SemaphoreType.DMA((N_LAYERS, N_DEV - 1)),
        ],
        compiler_params=pltpu.CompilerParams(collective_id=0),
    )(x, Win0, Wout0, Win1, Wout1, Win2, Wout2)


# device time: 20022 ns/iter; 1.0601x vs baseline; 1.0601x over previous
import jax
import jax.numpy as jnp
from jax import lax
from jax.experimental import pallas as pl
from jax.experimental.pallas import tpu as pltpu

N_DEV = 4
N_LAYERS = 3
B = 128
D = 128
H = 256
RB = B // N_DEV

SEND_ORDER = (2, 1, 3)
RECV_ORDER = (1, 3, 2)


def kernel(x, Win0, Wout0, Win1, Wout1, Win2, Wout2):
    def body(x_ref, win0_ref, wout0_ref, win1_ref, wout1_ref,
             win2_ref, wout2_ref, out_ref,
             partial_ref, comm_ref, rs_ref, send_sems, recv_sems):
        my = lax.axis_index("i")

        barrier_sem = pltpu.get_barrier_semaphore()
        for d in range(1, N_DEV):
            peer = lax.rem(my + d, N_DEV)
            pl.semaphore_signal(
                barrier_sem, inc=1,
                device_id=(peer,), device_id_type=pl.DeviceIdType.MESH,
            )
        pl.semaphore_wait(barrier_sem, N_DEV - 1)

        win_refs = [win0_ref, win1_ref, win2_ref]
        wout_refs = [wout0_ref, wout1_ref, wout2_ref]

        pending_sends = []
        xv = x_ref[:, :]

        for r in range(N_LAYERS - 1):
            h = jnp.maximum(
                jnp.dot(xv, win_refs[r][:, :],
                        preferred_element_type=jnp.float32),
                0.0,
            )
            partial = jnp.dot(h, wout_refs[r][:, :],
                              preferred_element_type=jnp.float32)
            partial_ref[r] = partial

            rdmas = {}
            for d in SEND_ORDER:
                peer = lax.rem(my + d, N_DEV)
                rdma = pltpu.make_async_remote_copy(
                    src_ref=partial_ref.at[r],
                    dst_ref=comm_ref.at[r, d - 1],
                    send_sem=send_sems.at[r, d - 1],
                    recv_sem=recv_sems.at[r, d - 1],
                    device_id=(peer,),
                    device_id_type=pl.DeviceIdType.MESH,
                )
                rdma.start()
                rdmas[d] = rdma
            acc = partial
            for d in RECV_ORDER:
                rdmas[d].wait_recv()
                acc = acc + comm_ref[r, d - 1]
            pending_sends.extend(rdmas.values())
            xv = acc

        r = N_LAYERS - 1
        h = jnp.maximum(
            jnp.dot(xv, win_refs[r][:, :],
                    preferred_element_type=jnp.float32),
            0.0,
        )
        partial = jnp.dot(h, wout_refs[r][:, :],
                          preferred_element_type=jnp.float32)
        partial_ref[r] = partial

        rdmas = {}
        for d in SEND_ORDER:
            peer = lax.rem(my + d, N_DEV)
            rdma = pltpu.make_async_remote_copy(
                src_ref=partial_ref.at[r, pl.ds(peer * RB, RB), :],
                dst_ref=rs_ref.at[d - 1],
                send_sem=send_sems.at[r, d - 1],
                recv_sem=recv_sems.at[r, d - 1],
                device_id=(peer,),
                device_id_type=pl.DeviceIdType.MESH,
            )
            rdma.start()
            rdmas[d] = rdma
        acc = partial_ref[r, pl.ds(my * RB, RB), :]
        for d in RECV_ORDER:
            rdmas[d].wait_recv()
            acc = acc + rs_ref[d - 1]
        pending_sends.extend(rdmas.values())

        out_ref[:, :] = acc

        for rdma in pending_sends:
            rdma.wait_send()

    return pl.pallas_call(
        body,
        out_shape=jax.ShapeDtypeStruct((RB, D), jnp.float32),
        in_specs=[pl.BlockSpec(memory_space=pltpu.VMEM)] * 7,
        out_specs=pl.BlockSpec(memory_space=pltpu.VMEM),
        scratch_shapes=[
            pltpu.VMEM((N_LAYERS, B, D), jnp.float32),
            pltpu.VMEM((N_LAYERS - 1, N_DEV - 1, B, D), jnp.float32),
            pltpu.VMEM((N_DEV - 1, RB, D), jnp.float32),
            pltpu.SemaphoreType.DMA((N_LAYERS, N_DEV - 1)),
            pltpu.SemaphoreType.DMA((N_LAYERS, N_DEV - 1)),
        ],
        compiler_params=pltpu.CompilerParams(collective_id=0),
    )(x, Win0, Wout0, Win1, Wout1, Win2, Wout2)


# device time: 18490 ns/iter; 1.1479x vs baseline; 1.0829x over previous
import jax
import jax.numpy as jnp
from jax import lax
from jax.experimental import pallas as pl
from jax.experimental.pallas import tpu as pltpu

N_DEV = 4
N_LAYERS = 3
B = 128
D = 128
H = 256
RB = B // N_DEV

SEND_ORDER = (2, 1, 3)
RECV_ORDER = (1, 3, 2)


def kernel(x, Win0, Wout0, Win1, Wout1, Win2, Wout2):
    def body(x_ref, win0_ref, wout0_ref, win1_ref, wout1_ref,
             win2_ref, wout2_ref, out_ref,
             partial_ref, comm_ref, rs_ref, send_sems, recv_sems):
        my = lax.axis_index("i")

        barrier_sem = pltpu.get_barrier_semaphore()
        for d in range(1, N_DEV):
            peer = lax.rem(my + d, N_DEV)
            pl.semaphore_signal(
                barrier_sem, inc=1,
                device_id=(peer,), device_id_type=pl.DeviceIdType.MESH,
            )
        pl.semaphore_wait(barrier_sem, N_DEV - 1)

        win_refs = [win0_ref, win1_ref, win2_ref]
        wout_refs = [wout0_ref, wout1_ref, wout2_ref]

        pending_sends = []
        xv = x_ref[:, :].astype(jnp.bfloat16)

        for r in range(N_LAYERS - 1):
            h = jnp.maximum(
                jnp.dot(xv, win_refs[r][:, :].astype(jnp.bfloat16),
                        preferred_element_type=jnp.float32),
                0.0,
            ).astype(jnp.bfloat16)
            partial = jnp.dot(h, wout_refs[r][:, :].astype(jnp.bfloat16),
                              preferred_element_type=jnp.float32)
            partial_ref[r] = partial.astype(jnp.bfloat16)

            rdmas = {}
            for d in SEND_ORDER:
                peer = lax.rem(my + d, N_DEV)
                rdma = pltpu.make_async_remote_copy(
                    src_ref=partial_ref.at[r],
                    dst_ref=comm_ref.at[r, d - 1],
                    send_sem=send_sems.at[r, d - 1],
                    recv_sem=recv_sems.at[r, d - 1],
                    device_id=(peer,),
                    device_id_type=pl.DeviceIdType.MESH,
                )
                rdma.start()
                rdmas[d] = rdma
            acc = partial
            for d in RECV_ORDER:
                rdmas[d].wait_recv()
                acc = acc + comm_ref[r, d - 1].astype(jnp.float32)
            pending_sends.extend(rdmas.values())
            xv = acc.astype(jnp.bfloat16)

        r = N_LAYERS - 1
        h = jnp.maximum(
            jnp.dot(xv, win_refs[r][:, :].astype(jnp.bfloat16),
                    preferred_element_type=jnp.float32),
            0.0,
        ).astype(jnp.bfloat16)
        partial = jnp.dot(h, wout_refs[r][:, :].astype(jnp.bfloat16),
                          preferred_element_type=jnp.float32)
        partial_ref[r] = partial.astype(jnp.bfloat16)

        rdmas = {}
        for d in SEND_ORDER:
            peer = lax.rem(my + d, N_DEV)
            rdma = pltpu.make_async_remote_copy(
                src_ref=partial_ref.at[r, pl.ds(peer * RB, RB), :],
                dst_ref=rs_ref.at[d - 1],
                send_sem=send_sems.at[r, d - 1],
                recv_sem=recv_sems.at[r, d - 1],
                device_id=(peer,),
                device_id_type=pl.DeviceIdType.MESH,
            )
            rdma.start()
            rdmas[d] = rdma
        acc = partial_ref[r, pl.ds(my * RB, RB), :].astype(jnp.float32)
        for d in RECV_ORDER:
            rdmas[d].wait_recv()
            acc = acc + rs_ref[d - 1].astype(jnp.float32)
        pending_sends.extend(rdmas.values())

        out_ref[:, :] = acc

        for rdma in pending_sends:
            rdma.wait_send()

    return pl.pallas_call(
        body,
        out_shape=jax.ShapeDtypeStruct((RB, D), jnp.float32),
        in_specs=[pl.BlockSpec(memory_space=pltpu.VMEM)] * 7,
        out_specs=pl.BlockSpec(memory_space=pltpu.VMEM),
        scratch_shapes=[
            pltpu.VMEM((N_LAYERS, B, D), jnp.bfloat16),
            pltpu.VMEM((N_LAYERS - 1, N_DEV - 1, B, D), jnp.bfloat16),
            pltpu.VMEM((N_DEV - 1, RB, D), jnp.bfloat16),
            pltpu.SemaphoreType.DMA((N_LAYERS, N_DEV - 1)),
            pltpu.SemaphoreType.DMA((N_LAYERS, N_DEV - 1)),
        ],
        compiler_params=pltpu.CompilerParams(collective_id=0),
    )(x, Win0, Wout0, Win1, Wout1, Win2, Wout2)


# device time: 18237 ns/iter; 1.1638x vs baseline; 1.0139x over previous
import jax
import jax.numpy as jnp
from jax import lax
from jax.experimental import pallas as pl
from jax.experimental.pallas import tpu as pltpu

N_DEV = 4
N_LAYERS = 3
B = 128
D = 128
H = 256
RB = B // N_DEV

SEND_ORDER = (2, 1, 3)
RECV_ORDER = (1, 3, 2)

BF16 = jnp.bfloat16
F32 = jnp.float32


def kernel(x, Win0, Wout0, Win1, Wout1, Win2, Wout2):
    def body(x_ref, win0_ref, wout0_ref, win1_ref, wout1_ref,
             win2_ref, wout2_ref, out_ref,
             partial_ref, comm_ref, rs_ref, send_sems, recv_sems):
        my = lax.axis_index("i")

        barrier_sem = pltpu.get_barrier_semaphore()
        for d in range(1, N_DEV):
            peer = lax.rem(my + d, N_DEV)
            pl.semaphore_signal(
                barrier_sem, inc=1,
                device_id=(peer,), device_id_type=pl.DeviceIdType.MESH,
            )

        wins = [win0_ref[:, :].astype(BF16), win1_ref[:, :].astype(BF16),
                win2_ref[:, :].astype(BF16)]
        wouts = [wout0_ref[:, :].astype(BF16), wout1_ref[:, :].astype(BF16),
                 wout2_ref[:, :].astype(BF16)]

        def layer(xv, r):
            h = jnp.maximum(
                jnp.dot(xv, wins[r], preferred_element_type=F32), 0.0
            ).astype(BF16)
            return jnp.dot(h, wouts[r], preferred_element_type=F32)

        pending_sends = []
        xv = x_ref[:, :].astype(BF16)

        for r in range(N_LAYERS - 1):
            partial = layer(xv, r).astype(BF16)
            partial_ref[r] = partial
            if r == 0:
                pl.semaphore_wait(barrier_sem, N_DEV - 1)

            rdmas = {}
            for d in SEND_ORDER:
                peer = lax.rem(my + d, N_DEV)
                rdma = pltpu.make_async_remote_copy(
                    src_ref=partial_ref.at[r],
                    dst_ref=comm_ref.at[r, d - 1],
                    send_sem=send_sems.at[r, d - 1],
                    recv_sem=recv_sems.at[r, d - 1],
                    device_id=(peer,),
                    device_id_type=pl.DeviceIdType.MESH,
                )
                rdma.start()
                rdmas[d] = rdma
            acc = partial
            for d in RECV_ORDER:
                rdmas[d].wait_recv()
                acc = acc + comm_ref[r, d - 1]
            pending_sends.extend(rdmas.values())
            xv = acc

        r = N_LAYERS - 1
        partial_ref[r] = layer(xv, r).astype(BF16)

        rdmas = {}
        for d in SEND_ORDER:
            peer = lax.rem(my + d, N_DEV)
            rdma = pltpu.make_async_remote_copy(
                src_ref=partial_ref.at[r, pl.ds(peer * RB, RB), :],
                dst_ref=rs_ref.at[d - 1],
                send_sem=send_sems.at[r, d - 1],
                recv_sem=recv_sems.at[r, d - 1],
                device_id=(peer,),
                device_id_type=pl.DeviceIdType.MESH,
            )
            rdma.start()
            rdmas[d] = rdma
        acc = partial_ref[r, pl.ds(my * RB, RB), :].astype(F32)
        for d in RECV_ORDER:
            rdmas[d].wait_recv()
            acc = acc + rs_ref[d - 1].astype(F32)
        pending_sends.extend(rdmas.values())

        out_ref[:, :] = acc

        for rdma in pending_sends:
            rdma.wait_send()

    return pl.pallas_call(
        body,
        out_shape=jax.ShapeDtypeStruct((RB, D), F32),
        in_specs=[pl.BlockSpec(memory_space=pltpu.VMEM)] * 7,
        out_specs=pl.BlockSpec(memory_space=pltpu.VMEM),
        scratch_shapes=[
            pltpu.VMEM((N_LAYERS, B, D), BF16),
            pltpu.VMEM((N_LAYERS - 1, N_DEV - 1, B, D), BF16),
            pltpu.VMEM((N_DEV - 1, RB, D), BF16),
            pltpu.SemaphoreType.DMA((N_LAYERS, N_DEV - 1)),
            pltpu.SemaphoreType.DMA((N_LAYERS, N_DEV - 1)),
        ],
        compiler_params=pltpu.CompilerParams(collective_id=0),
    )(x, Win0, Wout0, Win1, Wout1, Win2, Wout2)
